# Initial kernel scaffold; baseline (speedup 1.0000x reference)
import jax
import jax.numpy as jnp
from jax.experimental import pallas as pl


def kernel(x, importance, W_all, b_all, neuron_emb):
    raise NotImplementedError("write your pallas kernel here")


# speedup vs baseline: 31.7029x; 31.7029x over previous
import jax
import jax.numpy as jnp
from jax.experimental import pallas as pl

D_MODEL = 1024
D_SPACE = 64
POOL_N = 2048
SEQ = 2048
TOPK = 8
TOK_BLOCK = 256
N_HEADS = 6


def _router_body(x_ref, w_ref, b_ref, emb_ref, out_ref):
    x = x_ref[...]
    emb = emb_ref[...]
    inv = 1.0 / (jnp.sqrt(jnp.sum(emb * emb, axis=-1, keepdims=True)) + 1e-12)
    embn = emb * inv

    w = w_ref[...]
    b = b_ref[0]
    h = jax.lax.dot_general(x, w, (((1,), (1,)), ((), ())),
                            preferred_element_type=jnp.float32) + b
    logits = jax.lax.dot_general(h, embn, (((1,), (1,)), ((), ())),
                                 preferred_element_type=jnp.float32)

    m = jnp.max(logits, axis=-1, keepdims=True)
    ex = jnp.exp(logits - m)
    z = jnp.sum(ex, axis=-1, keepdims=True)

    v = logits
    for _ in range(TOPK - 1):
        cur = jnp.max(v, axis=-1, keepdims=True)
        v = jnp.where(v >= cur, -jnp.inf, v)
    kth = jnp.max(v, axis=-1, keepdims=True)

    exm = jnp.where(logits >= kth, ex, 0.0)
    s = jnp.sum(exm, axis=-1, keepdims=True)
    out_ref[0, :, :] = exm / (s + 1e-8 * z)


def kernel(x, importance, W_all, b_all, neuron_emb):
    del importance
    xs = x.reshape(SEQ, D_MODEL)
    emb = neuron_emb[: 4 * POOL_N]
    b2 = b_all.reshape(N_HEADS, 1, D_SPACE)

    n_tb = SEQ // TOK_BLOCK

    out = pl.pallas_call(
        _router_body,
        grid=(n_tb, N_HEADS),
        in_specs=[
            pl.BlockSpec((TOK_BLOCK, D_MODEL), lambda t, h: (t, 0)),
            pl.BlockSpec((D_SPACE, D_MODEL), lambda t, h: (h, 0)),
            pl.BlockSpec((1, 1, D_SPACE), lambda t, h: (h, 0, 0)),
            pl.BlockSpec((POOL_N, D_SPACE), lambda t, h: (h - (h + 2) // 3, 0)),
        ],
        out_specs=pl.BlockSpec((1, TOK_BLOCK, POOL_N), lambda t, h: (h, t, 0)),
        out_shape=jax.ShapeDtypeStruct((N_HEADS, SEQ, POOL_N), jnp.float32),
    )(xs, W_all, b2, emb)

    return out.reshape(N_HEADS, 1, SEQ, POOL_N)
